# Initial kernel scaffold; baseline (speedup 1.0000x reference)
import jax
import jax.numpy as jnp
from jax.experimental import pallas as pl


def kernel(embeddings):
    raise NotImplementedError("write your pallas kernel here")


# speedup vs baseline: 2.1864x; 2.1864x over previous
import functools

import jax
import jax.numpy as jnp
from jax import lax
from jax.experimental import pallas as pl
from jax.experimental.pallas import tpu as pltpu
from jax.experimental.pallas import tpu_sc as plsc

N = 10000
D = 128
K = 20
KP = 32
L = 16
NC, NS = 2, 16
NW = NC * NS
ROWS_PER_W = 313
TOTAL_ROWS = NW * ROWS_PER_W
CHUNKS = N // L
CAND = 2048
NEG = -3.0e38
BIG = 2**30

BR = 400


def _norm_body(emb_ref, out_ref):
    x = emb_ref[...]
    sq = jnp.sum(x * x, axis=1, keepdims=True)
    out_ref[...] = x * lax.rsqrt(jnp.maximum(sq, 1e-12))


def _matmul_body(a_ref, b_ref, out_ref):
    out_ref[...] = lax.dot_general(
        a_ref[...], b_ref[...],
        (((1,), (1,)), ((), ())),
        preferred_element_type=jnp.float32,
    )


def _topk_sc_body(sim_hbm, outv_hbm, outi_hbm, rowbuf, cvals, cidx,
                  ovals, oidx):
    wid = lax.axis_index("s") * NC + lax.axis_index("c")
    base = wid * ROWS_PER_W
    iota16 = lax.iota(jnp.int32, L)
    neg16 = jnp.full((L,), NEG, jnp.float32)
    big16 = jnp.full((L,), BIG, jnp.int32)
    lane0 = iota16 == 0

    def row_body(r, _):
        g = base + r

        @pl.when(g < N)
        def _():
            pltpu.sync_copy(sim_hbm.at[g], rowbuf)

            def p1(c, carry):
                r1, r2 = carry
                v = rowbuf[pl.ds(c * L, L)]
                m = jnp.maximum(r1, v)
                r2 = jnp.maximum(r2, jnp.minimum(r1, v))
                return m, r2

            r1, r2 = lax.fori_loop(0, CHUNKS, p1, (neg16, neg16))
            t = jnp.min(r2)

            def p2(c, cnt):
                v = rowbuf[pl.ds(c * L, L)]
                m = v >= t
                ok = cnt <= CAND - L
                m = jnp.logical_and(m, ok)
                plsc.store_compressed(cvals.at[pl.ds(cnt, L)], v, mask=m)
                plsc.store_compressed(
                    cidx.at[pl.ds(cnt, L)], c * L + iota16, mask=m)
                return cnt + jnp.sum(m.astype(jnp.int32))

            cnt = lax.fori_loop(0, CHUNKS, p2, jnp.int32(0))
            cvals[pl.ds(cnt, L)] = neg16
            nv = (cnt + L - 1) // L

            def sel(k, carry):
                av0, av1, ai0, ai1 = carry

                def scan(j, c2):
                    bv, bp = c2
                    v = cvals[pl.ds(j * L, L)]
                    better = v > bv
                    bv = jnp.where(better, v, bv)
                    bp = jnp.where(better, j * L + iota16, bp)
                    return bv, bp

                bv, bp = lax.fori_loop(0, nv, scan, (neg16, big16))
                vmax = jnp.max(bv)
                pos = jnp.min(jnp.where(bv == vmax, bp, big16))
                pos16 = jnp.full((L,), pos, jnp.int32)
                idxv = plsc.load_gather(cidx, [pos16])
                plsc.store_scatter(cvals, [pos16], neg16, mask=lane0)
                av0 = jnp.where(iota16 == k, vmax, av0)
                av1 = jnp.where(iota16 == k - L, vmax, av1)
                ai0 = jnp.where(iota16 == k, idxv, ai0)
                ai1 = jnp.where(iota16 == k - L, idxv, ai1)
                return av0, av1, ai0, ai1

            av0, av1, ai0, ai1 = lax.fori_loop(
                0, K, sel, (neg16, neg16, big16, big16))
            ovals[pl.ds(r * KP, L)] = av0
            ovals[pl.ds(r * KP + L, L)] = av1
            oidx[pl.ds(r * KP, L)] = ai0
            oidx[pl.ds(r * KP + L, L)] = ai1

        return 0

    lax.fori_loop(0, ROWS_PER_W, row_body, 0)
    pltpu.sync_copy(ovals, outv_hbm.at[pl.ds(base * KP, ROWS_PER_W * KP)])
    pltpu.sync_copy(oidx, outi_hbm.at[pl.ds(base * KP, ROWS_PER_W * KP)])


def _build_topk_sc():
    return functools.partial(
        pl.kernel,
        out_type=[
            jax.ShapeDtypeStruct((TOTAL_ROWS * KP,), jnp.float32),
            jax.ShapeDtypeStruct((TOTAL_ROWS * KP,), jnp.int32),
        ],
        mesh=plsc.VectorSubcoreMesh(core_axis_name="c", subcore_axis_name="s",
                                    num_cores=NC, num_subcores=NS),
        compiler_params=pltpu.CompilerParams(needs_layout_passes=False),
        scratch_types=[
            pltpu.VMEM((N,), jnp.float32),
            pltpu.VMEM((CAND + L,), jnp.float32),
            pltpu.VMEM((CAND + L,), jnp.int32),
            pltpu.VMEM((ROWS_PER_W * KP,), jnp.float32),
            pltpu.VMEM((ROWS_PER_W * KP,), jnp.int32),
        ],
    )(_topk_sc_body)


def kernel(embeddings):
    norm = pl.pallas_call(
        _norm_body,
        out_shape=jax.ShapeDtypeStruct((N, D), jnp.float32),
    )(embeddings)

    sim = pl.pallas_call(
        _matmul_body,
        grid=(N // BR,),
        in_specs=[
            pl.BlockSpec((BR, D), lambda i: (i, 0)),
            pl.BlockSpec((N, D), lambda i: (0, 0)),
        ],
        out_specs=pl.BlockSpec((BR, N), lambda i: (i, 0)),
        out_shape=jax.ShapeDtypeStruct((N, N), jnp.float32),
    )(norm, norm)

    vflat, iflat = _build_topk_sc()(sim)
    vals = vflat.reshape(TOTAL_ROWS, KP)[:N, :K]
    idx = iflat.reshape(TOTAL_ROWS, KP)[:N, :K]
    return vals, idx


# speedup vs baseline: 4.4433x; 2.0323x over previous
import functools

import jax
import jax.numpy as jnp
from jax import lax
from jax.experimental import pallas as pl
from jax.experimental.pallas import tpu as pltpu
from jax.experimental.pallas import tpu_sc as plsc

N = 10000
D = 128
K = 20
KP = 32
L = 16
NC, NS = 2, 16
NW = NC * NS
ROWS_PER_W = 313
TOTAL_ROWS = NW * ROWS_PER_W
CHUNKS = N // L
HALF = 312
CAND = N + 2 * L
NEG = -3.0e38
BIG = 2**30

BR = 400


def _norm_body(emb_ref, out_ref):
    x = emb_ref[...]
    sq = jnp.sum(x * x, axis=1, keepdims=True)
    out_ref[...] = x * lax.rsqrt(jnp.maximum(sq, 1e-12))


def _matmul_body(a_ref, b_ref, out_ref):
    out_ref[...] = lax.dot_general(
        a_ref[...], b_ref[...],
        (((1,), (1,)), ((), ())),
        preferred_element_type=jnp.float32,
    )


def _topk_sc_body(sim_hbm, outv_hbm, outi_hbm, rowbuf, cvals, cidx,
                  ovals, oidx, sem0, sem1):
    wid = lax.axis_index("s") * NC + lax.axis_index("c")
    base = wid * ROWS_PER_W
    iota16 = lax.iota(jnp.int32, L)
    neg16 = jnp.full((L,), NEG, jnp.float32)
    big16 = jnp.full((L,), BIG, jnp.int32)
    lane0 = iota16 == 0
    sems = (sem0, sem1)

    def valid(r):
        return jnp.logical_and(r < ROWS_PER_W, base + r < N)

    def start(r, slot):
        @pl.when(valid(r))
        def _():
            pltpu.make_async_copy(
                sim_hbm.at[base + r],
                rowbuf.at[slot],
                sems[slot],
            ).start()

    def process(r, slot):
        @pl.when(valid(r))
        def _():
            pltpu.make_async_copy(
                sim_hbm.at[base + r],
                rowbuf.at[slot],
                sems[slot],
            ).wait()

            def p1(c, carry):
                a1, a2 = carry
                v1 = rowbuf[slot, pl.ds(c * L, L)]
                v2 = rowbuf[slot, pl.ds((c + HALF) * L, L)]
                return jnp.maximum(a1, v1), jnp.maximum(a2, v2)

            a1, a2 = lax.fori_loop(0, HALF, p1, (neg16, neg16),
                                   unroll=4)
            a2 = jnp.maximum(a2, rowbuf[slot, pl.ds((CHUNKS - 1) * L, L)])
            t = jnp.minimum(jnp.min(a1), jnp.min(a2))

            def p2(c, cnt):
                v = rowbuf[slot, pl.ds(c * L, L)]
                m = v >= t
                plsc.store_compressed(cvals.at[pl.ds(cnt, L)], v, mask=m)
                plsc.store_compressed(
                    cidx.at[pl.ds(cnt, L)], c * L + iota16, mask=m)
                return cnt + jnp.sum(m.astype(jnp.int32))

            cnt = lax.fori_loop(0, CHUNKS, p2, jnp.int32(0), unroll=5)
            cvals[pl.ds(cnt, L)] = neg16
            nv = (cnt + L - 1) // L

            def sel(k, carry):
                av0, av1, ai0, ai1 = carry

                def scan(j, c2):
                    bv, bp = c2
                    v = cvals[pl.ds(j * L, L)]
                    better = v > bv
                    bv = jnp.where(better, v, bv)
                    bp = jnp.where(better, j * L + iota16, bp)
                    return bv, bp

                bv, bp = lax.fori_loop(0, nv, scan, (neg16, big16))
                vmax = jnp.max(bv)
                pos = jnp.min(jnp.where(bv == vmax, bp, big16))
                pos16 = jnp.full((L,), pos, jnp.int32)
                idxv = plsc.load_gather(cidx, [pos16])
                plsc.store_scatter(cvals, [pos16], neg16, mask=lane0)
                av0 = jnp.where(iota16 == k, vmax, av0)
                av1 = jnp.where(iota16 == k - L, vmax, av1)
                ai0 = jnp.where(iota16 == k, idxv, ai0)
                ai1 = jnp.where(iota16 == k - L, idxv, ai1)
                return av0, av1, ai0, ai1

            av0, av1, ai0, ai1 = lax.fori_loop(
                0, K, sel, (neg16, neg16, big16, big16))
            ovals[pl.ds(r * KP, L)] = av0
            ovals[pl.ds(r * KP + L, L)] = av1
            oidx[pl.ds(r * KP, L)] = ai0
            oidx[pl.ds(r * KP + L, L)] = ai1

    start(0, 0)
    start(1, 1)

    def outer(i, _):
        r0 = i * 2
        process(r0, 0)
        start(r0 + 2, 0)
        process(r0 + 1, 1)
        start(r0 + 3, 1)
        return 0

    lax.fori_loop(0, (ROWS_PER_W + 1) // 2, outer, 0)
    pltpu.sync_copy(ovals, outv_hbm.at[pl.ds(base * KP, ROWS_PER_W * KP)])
    pltpu.sync_copy(oidx, outi_hbm.at[pl.ds(base * KP, ROWS_PER_W * KP)])


def _build_topk_sc():
    return functools.partial(
        pl.kernel,
        out_type=[
            jax.ShapeDtypeStruct((TOTAL_ROWS * KP,), jnp.float32),
            jax.ShapeDtypeStruct((TOTAL_ROWS * KP,), jnp.int32),
        ],
        mesh=plsc.VectorSubcoreMesh(core_axis_name="c", subcore_axis_name="s",
                                    num_cores=NC, num_subcores=NS),
        compiler_params=pltpu.CompilerParams(needs_layout_passes=False),
        scratch_types=[
            pltpu.VMEM((2, N), jnp.float32),
            pltpu.VMEM((CAND,), jnp.float32),
            pltpu.VMEM((CAND,), jnp.int32),
            pltpu.VMEM((ROWS_PER_W * KP,), jnp.float32),
            pltpu.VMEM((ROWS_PER_W * KP,), jnp.int32),
            pltpu.SemaphoreType.DMA,
            pltpu.SemaphoreType.DMA,
        ],
    )(_topk_sc_body)


def kernel(embeddings):
    norm = pl.pallas_call(
        _norm_body,
        out_shape=jax.ShapeDtypeStruct((N, D), jnp.float32),
    )(embeddings)

    sim = pl.pallas_call(
        _matmul_body,
        grid=(N // BR,),
        in_specs=[
            pl.BlockSpec((BR, D), lambda i: (i, 0)),
            pl.BlockSpec((N, D), lambda i: (0, 0)),
        ],
        out_specs=pl.BlockSpec((BR, N), lambda i: (i, 0)),
        out_shape=jax.ShapeDtypeStruct((N, N), jnp.float32),
    )(norm, norm)

    vflat, iflat = _build_topk_sc()(sim)
    vals = vflat.reshape(TOTAL_ROWS, KP)[:N, :K]
    idx = iflat.reshape(TOTAL_ROWS, KP)[:N, :K]
    return vals, idx


# speedup vs baseline: 9.4607x; 2.1292x over previous
import functools

import jax
import jax.numpy as jnp
from jax import lax
from jax.experimental import pallas as pl
from jax.experimental.pallas import tpu as pltpu
from jax.experimental.pallas import tpu_sc as plsc

N = 10000
D = 128
K = 20
KP = 32
L = 16
NC, NS = 2, 16
NW = NC * NS
ROWS_PER_W = 313
TOTAL_ROWS = NW * ROWS_PER_W
CHUNKS = N // L
U = 5
GROUPS = CHUNKS // U
CAND = N + 2 * L
FBMAX = 512
NEG = -3.0e38
BIG = 2**30
TSPEC_INIT = 3.0e38

BR = 400


def _norm_body(emb_ref, out_ref):
    x = emb_ref[...]
    sq = jnp.sum(x * x, axis=1, keepdims=True)
    out_ref[...] = x * lax.rsqrt(jnp.maximum(sq, 1e-12))


def _matmul_body(a_ref, b_ref, out_ref):
    out_ref[...] = lax.dot_general(
        a_ref[...], b_ref[...],
        (((1,), (1,)), ((), ())),
        preferred_element_type=jnp.float32,
    )


def _topk_sc_body(sim_hbm, outv_hbm, outi_hbm, rowbuf,
                  cv0, cv1, ci0, ci1, ovals, oidx, sf, si,
                  sem0, sem1):
    wid = lax.axis_index("s") * NC + lax.axis_index("c")
    base = wid * ROWS_PER_W
    iota16 = lax.iota(jnp.int32, L)
    neg16 = jnp.full((L,), NEG, jnp.float32)
    big16 = jnp.full((L,), BIG, jnp.int32)
    lane0 = iota16 == 0
    sems = (sem0, sem1)
    cvs = (cv0, cv1)
    cis = (ci0, ci1)

    def valid(r):
        return jnp.logical_and(r < ROWS_PER_W, base + r < N)

    def start(r, slot):
        @pl.when(valid(r))
        def _():
            pltpu.make_async_copy(
                sim_hbm.at[base + r], rowbuf.at[slot], sems[slot]
            ).start()

    def filter_group(slot, i, t, cnt, with_max, a1, a2):
        cvals, cidx = cvs[slot], cis[slot]
        c0 = i * U
        vs = [rowbuf[slot, pl.ds((c0 + j) * L, L)] for j in range(U)]
        if with_max:
            a1 = jnp.maximum(a1, jnp.maximum(vs[0], vs[1]))
            a2 = jnp.maximum(
                a2, jnp.maximum(vs[2], jnp.maximum(vs[3], vs[4])))
        ms = [v >= t for v in vs]
        ns = [plsc.all_reduce_population_count(m)[0] for m in ms]
        o = cnt
        for j in range(U):
            plsc.store_compressed(cvals.at[pl.ds(o, L)], vs[j],
                                  mask=ms[j])
            plsc.store_compressed(cidx.at[pl.ds(o, L)],
                                  (c0 + j) * L + iota16, mask=ms[j])
            o = o + ns[j]
        return o, a1, a2

    def stream_pass(r, slot):
        si[slot] = jnp.int32(0)

        @pl.when(valid(r))
        def _():
            pltpu.make_async_copy(
                sim_hbm.at[base + r], rowbuf.at[slot], sems[slot]
            ).wait()
            tspec = sf[0]

            def g1(i, carry):
                a1, a2, cnt = carry
                cnt, a1, a2 = filter_group(slot, i, tspec, cnt,
                                           True, a1, a2)
                return a1, a2, cnt

            a1, a2, cnt = lax.fori_loop(
                0, GROUPS, g1, (neg16, neg16, jnp.int32(0)))
            spec_ok = jnp.logical_and(cnt >= K, cnt <= FBMAX)
            si[slot] = cnt

            @pl.when(jnp.logical_not(spec_ok))
            def _():
                t_ex = jnp.minimum(jnp.min(a1), jnp.min(a2))

                def g2(i, cnt):
                    cnt, _, _ = filter_group(slot, i, t_ex, cnt,
                                             False, a1, a2)
                    return cnt

                si[slot] = lax.fori_loop(0, GROUPS, g2, jnp.int32(0))

            cvs[slot][pl.ds(si[slot], L)] = neg16

    def joint_select(r0):
        cnt_a = si[0]
        cnt_b = si[1]
        nva = (cnt_a + L - 1) // L
        nvb = (cnt_b + L - 1) // L
        nvm = jnp.maximum(nva, nvb)

        def sel(k, carry):
            (av0a, av1a, ai0a, ai1a,
             av0b, av1b, ai0b, ai1b, v10, v20) = carry

            def scan(j, c2):
                bva, bpa, bvb, bpb = c2
                pa = j * L + iota16
                va = cv0[pl.ds(j * L, L)]
                vb = cv1[pl.ds(j * L, L)]
                beta = jnp.logical_and(va > bva, j < nva)
                betb = jnp.logical_and(vb > bvb, j < nvb)
                bva = jnp.where(beta, va, bva)
                bpa = jnp.where(beta, pa, bpa)
                bvb = jnp.where(betb, vb, bvb)
                bpb = jnp.where(betb, pa, bpb)
                return bva, bpa, bvb, bpb

            bva, bpa, bvb, bpb = lax.fori_loop(
                0, nvm, scan, (neg16, big16, neg16, big16))
            vma = jnp.max(bva)
            vmb = jnp.max(bvb)
            posa = jnp.minimum(
                jnp.min(jnp.where(bva == vma, bpa, big16)), CAND - 1)
            posb = jnp.minimum(
                jnp.min(jnp.where(bvb == vmb, bpb, big16)), CAND - 1)
            pa16 = jnp.full((L,), posa, jnp.int32)
            pb16 = jnp.full((L,), posb, jnp.int32)
            idxa = plsc.load_gather(ci0, [pa16])
            idxb = plsc.load_gather(ci1, [pb16])
            plsc.store_scatter(cv0, [pa16], neg16, mask=lane0)
            plsc.store_scatter(cv1, [pb16], neg16, mask=lane0)
            mk0 = iota16 == k
            mk1 = iota16 == k - L
            av0a = jnp.where(mk0, vma, av0a)
            av1a = jnp.where(mk1, vma, av1a)
            ai0a = jnp.where(mk0, idxa, ai0a)
            ai1a = jnp.where(mk1, idxa, ai1a)
            av0b = jnp.where(mk0, vmb, av0b)
            av1b = jnp.where(mk1, vmb, av1b)
            ai0b = jnp.where(mk0, idxb, ai0b)
            ai1b = jnp.where(mk1, idxb, ai1b)
            v10 = jnp.where(k == 10, vmb, v10)
            v20 = jnp.where(k == K - 1, vmb, v20)
            return (av0a, av1a, ai0a, ai1a,
                    av0b, av1b, ai0b, ai1b, v10, v20)

        (av0a, av1a, ai0a, ai1a, av0b, av1b, ai0b, ai1b,
         v10, v20) = lax.fori_loop(
            0, K, sel,
            (neg16, neg16, big16, big16,
             neg16, neg16, big16, big16,
             jnp.float32(0), jnp.float32(0)))

        @pl.when(valid(r0))
        def _():
            ovals[pl.ds(r0 * KP, L)] = av0a
            ovals[pl.ds(r0 * KP + L, L)] = av1a
            oidx[pl.ds(r0 * KP, L)] = ai0a
            oidx[pl.ds(r0 * KP + L, L)] = ai1a

        @pl.when(valid(r0 + 1))
        def _():
            ovals[pl.ds((r0 + 1) * KP, L)] = av0b
            ovals[pl.ds((r0 + 1) * KP + L, L)] = av1b
            oidx[pl.ds((r0 + 1) * KP, L)] = ai0b
            oidx[pl.ds((r0 + 1) * KP + L, L)] = ai1b
            sf[0] = 2.0 * v20 - v10

    sf[0] = jnp.float32(TSPEC_INIT)
    start(0, 0)
    start(1, 1)

    def outer(i, _):
        r0 = i * 2
        stream_pass(r0, 0)
        start(r0 + 2, 0)
        stream_pass(r0 + 1, 1)
        start(r0 + 3, 1)
        joint_select(r0)
        return 0

    lax.fori_loop(0, (ROWS_PER_W + 1) // 2, outer, 0)
    pltpu.sync_copy(ovals, outv_hbm.at[pl.ds(base * KP, ROWS_PER_W * KP)])
    pltpu.sync_copy(oidx, outi_hbm.at[pl.ds(base * KP, ROWS_PER_W * KP)])


def _build_topk_sc():
    return functools.partial(
        pl.kernel,
        out_type=[
            jax.ShapeDtypeStruct((TOTAL_ROWS * KP,), jnp.float32),
            jax.ShapeDtypeStruct((TOTAL_ROWS * KP,), jnp.int32),
        ],
        mesh=plsc.VectorSubcoreMesh(core_axis_name="c", subcore_axis_name="s",
                                    num_cores=NC, num_subcores=NS),
        compiler_params=pltpu.CompilerParams(needs_layout_passes=False),
        scratch_types=[
            pltpu.VMEM((2, N), jnp.float32),
            pltpu.VMEM((CAND,), jnp.float32),
            pltpu.VMEM((CAND,), jnp.float32),
            pltpu.VMEM((CAND,), jnp.int32),
            pltpu.VMEM((CAND,), jnp.int32),
            pltpu.VMEM((ROWS_PER_W * KP,), jnp.float32),
            pltpu.VMEM((ROWS_PER_W * KP,), jnp.int32),
            pltpu.SMEM((1,), jnp.float32),
            pltpu.SMEM((2,), jnp.int32),
            pltpu.SemaphoreType.DMA,
            pltpu.SemaphoreType.DMA,
        ],
    )(_topk_sc_body)


def kernel(embeddings):
    norm = pl.pallas_call(
        _norm_body,
        out_shape=jax.ShapeDtypeStruct((N, D), jnp.float32),
    )(embeddings)

    sim = pl.pallas_call(
        _matmul_body,
        grid=(N // BR,),
        in_specs=[
            pl.BlockSpec((BR, D), lambda i: (i, 0)),
            pl.BlockSpec((N, D), lambda i: (0, 0)),
        ],
        out_specs=pl.BlockSpec((BR, N), lambda i: (i, 0)),
        out_shape=jax.ShapeDtypeStruct((N, N), jnp.float32),
    )(norm, norm)

    vflat, iflat = _build_topk_sc()(sim)
    vals = vflat.reshape(TOTAL_ROWS, KP)[:N, :K]
    idx = iflat.reshape(TOTAL_ROWS, KP)[:N, :K]
    return vals, idx


# speedup vs baseline: 10.5575x; 1.1159x over previous
import functools

import jax
import jax.numpy as jnp
from jax import lax
from jax.experimental import pallas as pl
from jax.experimental.pallas import tpu as pltpu
from jax.experimental.pallas import tpu_sc as plsc

N = 10000
D = 128
K = 20
KP = 32
L = 16
NC, NS = 2, 16
NW = NC * NS
ROWS_PER_W = 313
TOTAL_ROWS = NW * ROWS_PER_W
CHUNKS = N // L
U = 5
GROUPS = CHUNKS // U
CAND = N + 2 * L
FBMAX = 512
NEG = -3.0e38
BIG = 2**30
TSPEC_INIT = 3.0e38

BR = 400


def _norm_body(emb_ref, out_ref):
    x = emb_ref[...]
    sq = jnp.sum(x * x, axis=1, keepdims=True)
    out_ref[...] = x * lax.rsqrt(jnp.maximum(sq, 1e-12))


def _matmul_body(a_ref, b_ref, out_ref):
    out_ref[...] = lax.dot_general(
        a_ref[...], b_ref[...],
        (((1,), (1,)), ((), ())),
        preferred_element_type=jnp.float32,
    )


def _topk_sc_body(sim_hbm, outv_hbm, outi_hbm,
                  rb0, rb1, rb2, rb3, cv0, cv1, ci0, ci1,
                  ovals, oidx, sf, si,
                  sem0, sem1, sem2, sem3):
    wid = lax.axis_index("s") * NC + lax.axis_index("c")
    base = wid * ROWS_PER_W
    iota16 = lax.iota(jnp.int32, L)
    neg16 = jnp.full((L,), NEG, jnp.float32)
    big16 = jnp.full((L,), BIG, jnp.int32)
    lane0 = iota16 == 0
    rbs = (rb0, rb1, rb2, rb3)
    sems = (sem0, sem1, sem2, sem3)

    def valid(r):
        return jnp.logical_and(r < ROWS_PER_W, base + r < N)

    def start(r, slot):
        @pl.when(valid(r))
        def _():
            pltpu.make_async_copy(
                sim_hbm.at[base + r], rbs[slot], sems[slot]
            ).start()

    def filter_group(rb, ci, i, t, cnt, with_max, a1, a2):
        c0 = i * U
        vs = [rb[pl.ds((c0 + j) * L, L)] for j in range(U)]
        if with_max:
            a1 = jnp.maximum(a1, jnp.maximum(vs[0], vs[1]))
            a2 = jnp.maximum(
                a2, jnp.maximum(vs[2], jnp.maximum(vs[3], vs[4])))
        ms = [v >= t for v in vs]
        ns = [plsc.all_reduce_population_count(m)[0] for m in ms]
        o = cnt
        for j in range(U):
            plsc.store_compressed(ci.at[pl.ds(o, L)],
                                  (c0 + j) * L + iota16, mask=ms[j])
            o = o + ns[j]
        return o, a1, a2

    def stream_pass(r, slot, half):
        rb, sem = rbs[slot], sems[slot]
        cv, ci = (cv0, ci0) if half == 0 else (cv1, ci1)
        si[half] = jnp.int32(0)

        @pl.when(valid(r))
        def _():
            pltpu.make_async_copy(
                sim_hbm.at[base + r], rb, sem
            ).wait()
            tspec = sf[0]

            def g1(i, carry):
                a1, a2, cnt = carry
                cnt, a1, a2 = filter_group(rb, ci, i, tspec, cnt,
                                           True, a1, a2)
                return a1, a2, cnt

            a1, a2, cnt = lax.fori_loop(
                0, GROUPS, g1, (neg16, neg16, jnp.int32(0)))
            spec_ok = jnp.logical_and(cnt >= K, cnt <= FBMAX)
            si[half] = cnt

            @pl.when(jnp.logical_not(spec_ok))
            def _():
                t_ex = jnp.minimum(jnp.min(a1), jnp.min(a2))

                def g2(i, cnt):
                    cnt, _, _ = filter_group(rb, ci, i, t_ex, cnt,
                                             False, a1, a2)
                    return cnt

                si[half] = lax.fori_loop(0, GROUPS, g2, jnp.int32(0))

            cnt2 = si[half]
            nv = (cnt2 + L - 1) // L
            ci[pl.ds(cnt2, L)] = jnp.zeros((L,), jnp.int32)

            def stage(j, _):
                iv = ci[pl.ds(j * L, L)]
                cv[pl.ds(j * L, L)] = plsc.load_gather(rb, [iv])
                return 0

            lax.fori_loop(0, nv, stage, 0)
            cv[pl.ds(cnt2, L)] = neg16

    def joint_select(r0):
        cnt_a = si[0]
        cnt_b = si[1]
        nva = (cnt_a + L - 1) // L
        nvb = (cnt_b + L - 1) // L
        nvm = jnp.maximum(nva, nvb)

        def sel(k, carry):
            (av0a, av1a, ai0a, ai1a,
             av0b, av1b, ai0b, ai1b, v10, v20) = carry

            def scan(j, c2):
                bva, bpa, bvb, bpb = c2
                pa = j * L + iota16
                va = cv0[pl.ds(j * L, L)]
                vb = cv1[pl.ds(j * L, L)]
                beta = jnp.logical_and(va > bva, j < nva)
                betb = jnp.logical_and(vb > bvb, j < nvb)
                bva = jnp.where(beta, va, bva)
                bpa = jnp.where(beta, pa, bpa)
                bvb = jnp.where(betb, vb, bvb)
                bpb = jnp.where(betb, pa, bpb)
                return bva, bpa, bvb, bpb

            bva, bpa, bvb, bpb = lax.fori_loop(
                0, nvm, scan, (neg16, big16, neg16, big16))
            vma = jnp.max(bva)
            vmb = jnp.max(bvb)
            posa = jnp.minimum(
                jnp.min(jnp.where(bva == vma, bpa, big16)), CAND - 1)
            posb = jnp.minimum(
                jnp.min(jnp.where(bvb == vmb, bpb, big16)), CAND - 1)
            pa16 = jnp.full((L,), posa, jnp.int32)
            pb16 = jnp.full((L,), posb, jnp.int32)
            idxa = plsc.load_gather(ci0, [pa16])
            idxb = plsc.load_gather(ci1, [pb16])
            plsc.store_scatter(cv0, [pa16], neg16, mask=lane0)
            plsc.store_scatter(cv1, [pb16], neg16, mask=lane0)
            mk0 = iota16 == k
            mk1 = iota16 == k - L
            av0a = jnp.where(mk0, vma, av0a)
            av1a = jnp.where(mk1, vma, av1a)
            ai0a = jnp.where(mk0, idxa, ai0a)
            ai1a = jnp.where(mk1, idxa, ai1a)
            av0b = jnp.where(mk0, vmb, av0b)
            av1b = jnp.where(mk1, vmb, av1b)
            ai0b = jnp.where(mk0, idxb, ai0b)
            ai1b = jnp.where(mk1, idxb, ai1b)
            v10 = jnp.where(k == 10, vmb, v10)
            v20 = jnp.where(k == K - 1, vmb, v20)
            return (av0a, av1a, ai0a, ai1a,
                    av0b, av1b, ai0b, ai1b, v10, v20)

        (av0a, av1a, ai0a, ai1a, av0b, av1b, ai0b, ai1b,
         v10, v20) = lax.fori_loop(
            0, K, sel,
            (neg16, neg16, big16, big16,
             neg16, neg16, big16, big16,
             jnp.float32(0), jnp.float32(0)))

        @pl.when(valid(r0))
        def _():
            ovals[pl.ds(r0 * KP, L)] = av0a
            ovals[pl.ds(r0 * KP + L, L)] = av1a
            oidx[pl.ds(r0 * KP, L)] = ai0a
            oidx[pl.ds(r0 * KP + L, L)] = ai1a

        @pl.when(valid(r0 + 1))
        def _():
            ovals[pl.ds((r0 + 1) * KP, L)] = av0b
            ovals[pl.ds((r0 + 1) * KP + L, L)] = av1b
            oidx[pl.ds((r0 + 1) * KP, L)] = ai0b
            oidx[pl.ds((r0 + 1) * KP + L, L)] = ai1b
            sf[0] = 2.0 * v20 - v10

    sf[0] = jnp.float32(TSPEC_INIT)
    for s in range(4):
        start(s, s)

    def outer(i, _):
        r0 = i * 4
        stream_pass(r0, 0, 0)
        stream_pass(r0 + 1, 1, 1)
        start(r0 + 4, 0)
        start(r0 + 5, 1)
        joint_select(r0)
        stream_pass(r0 + 2, 2, 0)
        stream_pass(r0 + 3, 3, 1)
        start(r0 + 6, 2)
        start(r0 + 7, 3)
        joint_select(r0 + 2)
        return 0

    lax.fori_loop(0, (ROWS_PER_W + 3) // 4, outer, 0)
    pltpu.sync_copy(ovals, outv_hbm.at[pl.ds(base * KP, ROWS_PER_W * KP)])
    pltpu.sync_copy(oidx, outi_hbm.at[pl.ds(base * KP, ROWS_PER_W * KP)])


def _build_topk_sc():
    return functools.partial(
        pl.kernel,
        out_type=[
            jax.ShapeDtypeStruct((TOTAL_ROWS * KP,), jnp.float32),
            jax.ShapeDtypeStruct((TOTAL_ROWS * KP,), jnp.int32),
        ],
        mesh=plsc.VectorSubcoreMesh(core_axis_name="c", subcore_axis_name="s",
                                    num_cores=NC, num_subcores=NS),
        compiler_params=pltpu.CompilerParams(needs_layout_passes=False),
        scratch_types=[
            pltpu.VMEM((N,), jnp.float32),
            pltpu.VMEM((N,), jnp.float32),
            pltpu.VMEM((N,), jnp.float32),
            pltpu.VMEM((N,), jnp.float32),
            pltpu.VMEM((CAND,), jnp.float32),
            pltpu.VMEM((CAND,), jnp.float32),
            pltpu.VMEM((CAND,), jnp.int32),
            pltpu.VMEM((CAND,), jnp.int32),
            pltpu.VMEM((ROWS_PER_W * KP,), jnp.float32),
            pltpu.VMEM((ROWS_PER_W * KP,), jnp.int32),
            pltpu.SMEM((1,), jnp.float32),
            pltpu.SMEM((2,), jnp.int32),
            pltpu.SemaphoreType.DMA,
            pltpu.SemaphoreType.DMA,
            pltpu.SemaphoreType.DMA,
            pltpu.SemaphoreType.DMA,
        ],
    )(_topk_sc_body)


def kernel(embeddings):
    norm = pl.pallas_call(
        _norm_body,
        out_shape=jax.ShapeDtypeStruct((N, D), jnp.float32),
    )(embeddings)

    sim = pl.pallas_call(
        _matmul_body,
        grid=(N // BR,),
        in_specs=[
            pl.BlockSpec((BR, D), lambda i: (i, 0)),
            pl.BlockSpec((N, D), lambda i: (0, 0)),
        ],
        out_specs=pl.BlockSpec((BR, N), lambda i: (i, 0)),
        out_shape=jax.ShapeDtypeStruct((N, N), jnp.float32),
    )(norm, norm)

    vflat, iflat = _build_topk_sc()(sim)
    vals = vflat.reshape(TOTAL_ROWS, KP)[:N, :K]
    idx = iflat.reshape(TOTAL_ROWS, KP)[:N, :K]
    return vals, idx
